# Initial kernel scaffold; baseline (speedup 1.0000x reference)
import jax
import jax.numpy as jnp
from jax.experimental import pallas as pl


def kernel(x, edge_index, ptr, W0, b0, W1, b1, W2, b2, M0, mb0, M1, mb1, M2, mb2):
    raise NotImplementedError("write your pallas kernel here")


# speedup vs baseline: 7.2962x; 7.2962x over previous
import functools

import jax
import jax.numpy as jnp
from jax import lax
from jax.experimental import pallas as pl
from jax.experimental.pallas import tpu as pltpu
from jax.experimental.pallas import tpu_sc as plsc

N = 10000
E = 160000
G = 20
NPG = 500
SLOPE = 0.1

NC = 2
NS = 16
NW = NC * NS
EPW = E // NW
EB = 128
NB = EPW // EB
TAIL = EPW - NB * EB
NP = 10240
RPS = NP // NS
ZR = 128


def _lrelu(v):
    return jnp.where(v >= 0, v, SLOPE * v)


def _make_scatter(d):
    mesh = plsc.VectorSubcoreMesh(core_axis_name="c", subcore_axis_name="s")

    @functools.partial(
        pl.kernel,
        out_type=jax.ShapeDtypeStruct((2 * NP, d), jnp.float32),
        mesh=mesh,
        compiler_params=pltpu.CompilerParams(use_tc_tiling_on_sc=False),
        scratch_types=[
            pltpu.VMEM((EB,), jnp.int32),
            pltpu.VMEM((EB,), jnp.int32),
            pltpu.VMEM((EB, d), jnp.float32),
            pltpu.VMEM((TAIL,), jnp.int32),
            pltpu.VMEM((TAIL,), jnp.int32),
            pltpu.VMEM((TAIL, d), jnp.float32),
            pltpu.VMEM((ZR, d), jnp.float32),
            pltpu.VMEM_SHARED((NP, d), jnp.float32),
        ],
    )
    def k(g_hbm, row_hbm, col_hbm, out_hbm,
          rowb, colb, msg, rowt, colt, msgt, zbuf, acc):
        c = lax.axis_index("c")
        s = lax.axis_index("s")
        wid = s * NC + c

        def zrow(i, carry):
            for j in range(d // 16):
                zbuf[i, pl.ds(j * 16, 16)] = jnp.zeros((16,), jnp.float32)
            return carry

        lax.fori_loop(0, ZR, zrow, 0)
        for j in range(RPS // ZR):
            pltpu.sync_copy(zbuf, acc.at[pl.ds(s * RPS + j * ZR, ZR)])
        plsc.subcore_barrier()

        base = wid * EPW

        def batch(j, carry):
            off = base + j * EB
            pltpu.sync_copy(row_hbm.at[pl.ds(off, EB)], rowb)
            pltpu.sync_copy(col_hbm.at[pl.ds(off, EB)], colb)
            pltpu.sync_copy(g_hbm.at[rowb], msg)
            pltpu.sync_copy(msg, acc.at[colb], add=True)
            return carry

        lax.fori_loop(0, NB, batch, 0)

        if TAIL:
            off = base + NB * EB
            pltpu.sync_copy(row_hbm.at[pl.ds(off, TAIL)], rowt)
            pltpu.sync_copy(col_hbm.at[pl.ds(off, TAIL)], colt)
            pltpu.sync_copy(g_hbm.at[rowt], msgt)
            pltpu.sync_copy(msgt, acc.at[colt], add=True)

        plsc.subcore_barrier()
        pltpu.sync_copy(acc.at[pl.ds(s * RPS, RPS)],
                        out_hbm.at[pl.ds(c * NP + s * RPS, RPS)])

    return k


_scatter16 = _make_scatter(16)
_scatter32 = _make_scatter(32)
_scatter64 = _make_scatter(64)


_R = 2000


def _prep_body(p0, p1, x, w, dinv_ref, g_ref):
    deg = p0[:, 0:1] + p1[:, 0:1]
    dinv = jnp.where(deg > 0, lax.rsqrt(deg), 0.0)
    dinv_ref[...] = dinv
    g_ref[...] = dinv * jnp.dot(x[...], w[...],
                                preferred_element_type=jnp.float32)


def _prep(p0, p1, x, w):
    din, dw = x.shape[1], w.shape[1]
    return pl.pallas_call(
        _prep_body,
        grid=(N // _R,),
        in_specs=[
            pl.BlockSpec((_R, 16), lambda i: (i, 0)),
            pl.BlockSpec((_R, 16), lambda i: (i, 0)),
            pl.BlockSpec((_R, din), lambda i: (i, 0)),
            pl.BlockSpec((din, dw), lambda i: (0, 0)),
        ],
        out_specs=[
            pl.BlockSpec((_R, 1), lambda i: (i, 0)),
            pl.BlockSpec((_R, dw), lambda i: (i, 0)),
        ],
        out_shape=[
            jax.ShapeDtypeStruct((N, 1), jnp.float32),
            jax.ShapeDtypeStruct((N, dw), jnp.float32),
        ],
    )(p0, p1, x, w)


def _mid_body(q0, q1, dinv, b, w, g_ref):
    a = dinv[...] * (q0[...] + q1[...]) + b[...]
    a = _lrelu(a)
    g_ref[...] = dinv[...] * jnp.dot(a, w[...],
                                     preferred_element_type=jnp.float32)


def _mid(q0, q1, dinv, b, w):
    d, dw = w.shape
    return pl.pallas_call(
        _mid_body,
        grid=(N // _R,),
        in_specs=[
            pl.BlockSpec((_R, d), lambda i: (i, 0)),
            pl.BlockSpec((_R, d), lambda i: (i, 0)),
            pl.BlockSpec((_R, 1), lambda i: (i, 0)),
            pl.BlockSpec((1, d), lambda i: (0, 0)),
            pl.BlockSpec((d, dw), lambda i: (0, 0)),
        ],
        out_specs=pl.BlockSpec((_R, dw), lambda i: (i, 0)),
        out_shape=jax.ShapeDtypeStruct((N, dw), jnp.float32),
    )(q0, q1, dinv, b, w)


def _final_body(q0, q1, dinv, b2, m0, mb0, m1, mb1, m2, mb2, h_ref, vp_ref):
    h = dinv[...] * (q0[...] + q1[...]) + b2[...]
    h_ref[...] = h
    t = _lrelu(jnp.dot(h, m0[...], preferred_element_type=jnp.float32) + mb0[...])
    t = _lrelu(jnp.dot(t, m1[...], preferred_element_type=jnp.float32) + mb1[...])
    vp_ref[...] = jnp.dot(t, m2[...], preferred_element_type=jnp.float32) + mb2[...]


def _final(q0, q1, dinv, b2, m0, mb0, m1, mb1, m2, mb2):
    return pl.pallas_call(
        _final_body,
        grid=(N // _R,),
        in_specs=[
            pl.BlockSpec((_R, 64), lambda i: (i, 0)),
            pl.BlockSpec((_R, 64), lambda i: (i, 0)),
            pl.BlockSpec((_R, 1), lambda i: (i, 0)),
            pl.BlockSpec((1, 64), lambda i: (0, 0)),
            pl.BlockSpec((64, 64), lambda i: (0, 0)),
            pl.BlockSpec((1, 64), lambda i: (0, 0)),
            pl.BlockSpec((64, 64), lambda i: (0, 0)),
            pl.BlockSpec((1, 64), lambda i: (0, 0)),
            pl.BlockSpec((64, 4), lambda i: (0, 0)),
            pl.BlockSpec((1, 4), lambda i: (0, 0)),
        ],
        out_specs=[
            pl.BlockSpec((_R, 64), lambda i: (i, 0)),
            pl.BlockSpec((_R, 4), lambda i: (i, 0)),
        ],
        out_shape=[
            jax.ShapeDtypeStruct((N, 64), jnp.float32),
            jax.ShapeDtypeStruct((N, 4), jnp.float32),
        ],
    )(q0, q1, dinv, b2, m0, mb0, m1, mb1, m2, mb2)


_OB = 400
_VD = NPG * 4


def _outer_body(vr, vc, o_ref):
    i = pl.program_id(1)
    rows = vr[0, pl.ds(i * _OB, _OB), :]
    o_ref[0] = rows * vc[0]


def _outer(vcol, vrow):
    return pl.pallas_call(
        _outer_body,
        grid=(G, _VD // _OB),
        in_specs=[
            pl.BlockSpec((1, _VD, 1), lambda k, i: (k, 0, 0)),
            pl.BlockSpec((1, 1, _VD), lambda k, i: (k, 0, 0)),
        ],
        out_specs=pl.BlockSpec((1, _OB, _VD), lambda k, i: (k, i, 0)),
        out_shape=jax.ShapeDtypeStruct((G, _VD, _VD), jnp.float32),
    )(vcol, vrow)


@jax.jit
def kernel(x, edge_index, ptr, W0, b0, W1, b1, W2, b2, M0, mb0, M1, mb1, M2, mb2):
    row = edge_index[0]
    col = edge_index[1]

    ones = jnp.ones((N, 16), jnp.float32)
    degp = _scatter16(ones, row, col)
    dinv, g0 = _prep(degp[:N], degp[NP:NP + N], x, W0)

    p = _scatter32(g0, row, col)
    g1 = _mid(p[:N], p[NP:NP + N], dinv, b0.reshape(1, -1), W1)

    p = _scatter32(g1, row, col)
    g2 = _mid(p[:N], p[NP:NP + N], dinv, b1.reshape(1, -1), W2)

    p = _scatter64(g2, row, col)
    h, vp = _final(p[:N], p[NP:NP + N], dinv, b2.reshape(1, -1),
                   M0, mb0.reshape(1, -1), M1, mb1.reshape(1, -1),
                   M2, mb2.reshape(1, -1))

    Xs = _outer(vp.reshape(G, _VD, 1), vp.reshape(G, 1, _VD))
    return h, Xs


# speedup vs baseline: 11.9517x; 1.6381x over previous
import functools

import jax
import jax.numpy as jnp
from jax import lax
from jax.experimental import pallas as pl
from jax.experimental.pallas import tpu as pltpu
from jax.experimental.pallas import tpu_sc as plsc

N = 10000
E = 160000
G = 20
NPG = 500
SLOPE = 0.1

NC = 2
NS = 16
NW = NC * NS
EPW = E // NW
NP = 10240
RPS = NP // NS
ZR = 128
EBW = 500
NTR = E // EBW
TPT = NTR // NW


def _lrelu(v):
    return jnp.where(v >= 0, v, SLOPE * v)


def _zero_acc(zbuf, acc, s, d):
    def zrow(i, carry):
        for j in range(d // 16):
            zbuf[i, pl.ds(j * 16, 16)] = jnp.zeros((16,), jnp.float32)
        return carry

    lax.fori_loop(0, ZR, zrow, 0)
    for j in range(RPS // ZR):
        pltpu.sync_copy(zbuf, acc.at[pl.ds(s * RPS + j * ZR, ZR)])


def _flush_acc(acc, out_hbm, c, s):
    pltpu.sync_copy(acc.at[pl.ds(s * RPS, RPS)],
                    out_hbm.at[pl.ds(c * NP + s * RPS, RPS)])


def _make_scatter(d):
    mesh = plsc.VectorSubcoreMesh(core_axis_name="c", subcore_axis_name="s")

    @functools.partial(
        pl.kernel,
        out_type=jax.ShapeDtypeStruct((2 * NP, d), jnp.float32),
        mesh=mesh,
        compiler_params=pltpu.CompilerParams(use_tc_tiling_on_sc=False),
        scratch_types=[
            pltpu.VMEM((TPT, EBW), jnp.int32),
            pltpu.VMEM((TPT, EBW), jnp.int32),
            pltpu.VMEM((EBW, d), jnp.float32),
            pltpu.VMEM((EBW, d), jnp.float32),
            pltpu.VMEM((ZR, d), jnp.float32),
            pltpu.VMEM_SHARED((NP, d), jnp.float32),
            pltpu.SemaphoreType.DMA,
            pltpu.SemaphoreType.DMA,
        ],
    )
    def k(g_hbm, rowm_hbm, colm_hbm, out_hbm,
          rowb, colb, msg0, msg1, zbuf, acc, sem0, sem1):
        c = lax.axis_index("c")
        s = lax.axis_index("s")
        wid = s * NC + c
        msg = (msg0, msg1)
        sem = (sem0, sem1)

        _zero_acc(zbuf, acc, s, d)

        br = wid * TPT
        pltpu.sync_copy(rowm_hbm.at[pl.ds(br, TPT)], rowb)
        pltpu.sync_copy(colm_hbm.at[pl.ds(br, TPT)], colb)
        plsc.subcore_barrier()

        def start_g(st, b):
            pltpu.async_copy(g_hbm.at[rowb.at[st]], msg[b], sem[b])

        def wait_g(b):
            pltpu.make_async_copy(g_hbm.at[pl.ds(0, EBW)], msg[b],
                                  sem[b]).wait()

        start_g(0, 0)

        def pair(t, carry):
            for b in range(2):
                st = 2 * t + b
                wait_g(b)

                @pl.when(st + 1 < TPT)
                def _():
                    start_g(st + 1, 1 - b)

                pltpu.sync_copy(msg[b], acc.at[colb.at[st]], add=True)
            return carry

        lax.fori_loop(0, TPT // 2, pair, 0)

        plsc.subcore_barrier()
        _flush_acc(acc, out_hbm, c, s)

    return k


def _make_deg():
    d = 16
    mesh = plsc.VectorSubcoreMesh(core_axis_name="c", subcore_axis_name="s")

    @functools.partial(
        pl.kernel,
        out_type=jax.ShapeDtypeStruct((2 * NP, d), jnp.float32),
        mesh=mesh,
        compiler_params=pltpu.CompilerParams(use_tc_tiling_on_sc=False),
        scratch_types=[
            pltpu.VMEM((TPT, EBW), jnp.int32),
            pltpu.VMEM((EBW, d), jnp.float32),
            pltpu.VMEM((ZR, d), jnp.float32),
            pltpu.VMEM_SHARED((NP, d), jnp.float32),
        ],
    )
    def k(colm_hbm, out_hbm, colb, ones, zbuf, acc):
        c = lax.axis_index("c")
        s = lax.axis_index("s")
        wid = s * NC + c

        def orow(i, carry):
            ones[i, pl.ds(0, 16)] = jnp.ones((16,), jnp.float32)
            return carry

        lax.fori_loop(0, EBW, orow, 0)
        _zero_acc(zbuf, acc, s, d)

        br = wid * TPT
        pltpu.sync_copy(colm_hbm.at[pl.ds(br, TPT)], colb)
        plsc.subcore_barrier()

        def step(st, carry):
            pltpu.sync_copy(ones, acc.at[colb.at[st]], add=True)
            return carry

        lax.fori_loop(0, TPT, step, 0)

        plsc.subcore_barrier()
        _flush_acc(acc, out_hbm, c, s)

    return k


_deg = _make_deg()
_scatter32 = _make_scatter(32)
_scatter64 = _make_scatter(64)


_R = 2000


def _prep_body(p0, p1, x, w, dinv_ref, g_ref):
    deg = p0[:, 0:1] + p1[:, 0:1]
    dinv = jnp.where(deg > 0, lax.rsqrt(deg), 0.0)
    dinv_ref[...] = dinv
    g_ref[...] = dinv * jnp.dot(x[...], w[...],
                                preferred_element_type=jnp.float32)


def _prep(p0, p1, x, w):
    din, dw = x.shape[1], w.shape[1]
    return pl.pallas_call(
        _prep_body,
        grid=(N // _R,),
        in_specs=[
            pl.BlockSpec((_R, 16), lambda i: (i, 0)),
            pl.BlockSpec((_R, 16), lambda i: (i, 0)),
            pl.BlockSpec((_R, din), lambda i: (i, 0)),
            pl.BlockSpec((din, dw), lambda i: (0, 0)),
        ],
        out_specs=[
            pl.BlockSpec((_R, 1), lambda i: (i, 0)),
            pl.BlockSpec((_R, dw), lambda i: (i, 0)),
        ],
        out_shape=[
            jax.ShapeDtypeStruct((N, 1), jnp.float32),
            jax.ShapeDtypeStruct((N, dw), jnp.float32),
        ],
    )(p0, p1, x, w)


def _mid_body(q0, q1, dinv, b, w, g_ref):
    a = dinv[...] * (q0[...] + q1[...]) + b[...]
    a = _lrelu(a)
    g_ref[...] = dinv[...] * jnp.dot(a, w[...],
                                     preferred_element_type=jnp.float32)


def _mid(q0, q1, dinv, b, w):
    d, dw = w.shape
    return pl.pallas_call(
        _mid_body,
        grid=(N // _R,),
        in_specs=[
            pl.BlockSpec((_R, d), lambda i: (i, 0)),
            pl.BlockSpec((_R, d), lambda i: (i, 0)),
            pl.BlockSpec((_R, 1), lambda i: (i, 0)),
            pl.BlockSpec((1, d), lambda i: (0, 0)),
            pl.BlockSpec((d, dw), lambda i: (0, 0)),
        ],
        out_specs=pl.BlockSpec((_R, dw), lambda i: (i, 0)),
        out_shape=jax.ShapeDtypeStruct((N, dw), jnp.float32),
    )(q0, q1, dinv, b, w)


def _final_body(q0, q1, dinv, b2, m0, mb0, m1, mb1, m2, mb2, h_ref, vp_ref):
    h = dinv[...] * (q0[...] + q1[...]) + b2[...]
    h_ref[...] = h
    t = _lrelu(jnp.dot(h, m0[...], preferred_element_type=jnp.float32) + mb0[...])
    t = _lrelu(jnp.dot(t, m1[...], preferred_element_type=jnp.float32) + mb1[...])
    vp_ref[...] = jnp.dot(t, m2[...], preferred_element_type=jnp.float32) + mb2[...]


def _final(q0, q1, dinv, b2, m0, mb0, m1, mb1, m2, mb2):
    return pl.pallas_call(
        _final_body,
        grid=(N // _R,),
        in_specs=[
            pl.BlockSpec((_R, 64), lambda i: (i, 0)),
            pl.BlockSpec((_R, 64), lambda i: (i, 0)),
            pl.BlockSpec((_R, 1), lambda i: (i, 0)),
            pl.BlockSpec((1, 64), lambda i: (0, 0)),
            pl.BlockSpec((64, 64), lambda i: (0, 0)),
            pl.BlockSpec((1, 64), lambda i: (0, 0)),
            pl.BlockSpec((64, 64), lambda i: (0, 0)),
            pl.BlockSpec((1, 64), lambda i: (0, 0)),
            pl.BlockSpec((64, 4), lambda i: (0, 0)),
            pl.BlockSpec((1, 4), lambda i: (0, 0)),
        ],
        out_specs=[
            pl.BlockSpec((_R, 64), lambda i: (i, 0)),
            pl.BlockSpec((_R, 4), lambda i: (i, 0)),
        ],
        out_shape=[
            jax.ShapeDtypeStruct((N, 64), jnp.float32),
            jax.ShapeDtypeStruct((N, 4), jnp.float32),
        ],
    )(q0, q1, dinv, b2, m0, mb0, m1, mb1, m2, mb2)


_OB = 400
_VD = NPG * 4


def _outer_body(vr, vc, o_ref):
    i = pl.program_id(1)
    rows = vr[0, pl.ds(i * _OB, _OB), :]
    o_ref[0] = rows * vc[0]


def _outer(vcol, vrow):
    return pl.pallas_call(
        _outer_body,
        grid=(G, _VD // _OB),
        in_specs=[
            pl.BlockSpec((1, _VD, 1), lambda k, i: (k, 0, 0)),
            pl.BlockSpec((1, 1, _VD), lambda k, i: (k, 0, 0)),
        ],
        out_specs=pl.BlockSpec((1, _OB, _VD), lambda k, i: (k, i, 0)),
        out_shape=jax.ShapeDtypeStruct((G, _VD, _VD), jnp.float32),
    )(vcol, vrow)


@jax.jit
def kernel(x, edge_index, ptr, W0, b0, W1, b1, W2, b2, M0, mb0, M1, mb1, M2, mb2):
    rowm = edge_index[0].reshape(NTR, EBW)
    colm = edge_index[1].reshape(NTR, EBW)

    degp = _deg(colm)
    dinv, g0 = _prep(degp[:N], degp[NP:NP + N], x, W0)

    p = _scatter32(g0, rowm, colm)
    g1 = _mid(p[:N], p[NP:NP + N], dinv, b0.reshape(1, -1), W1)

    p = _scatter32(g1, rowm, colm)
    g2 = _mid(p[:N], p[NP:NP + N], dinv, b1.reshape(1, -1), W2)

    p = _scatter64(g2, rowm, colm)
    h, vp = _final(p[:N], p[NP:NP + N], dinv, b2.reshape(1, -1),
                   M0, mb0.reshape(1, -1), M1, mb1.reshape(1, -1),
                   M2, mb2.reshape(1, -1))

    Xs = _outer(vp.reshape(G, _VD, 1), vp.reshape(G, 1, _VD))
    return h, Xs


# speedup vs baseline: 12.8929x; 1.0787x over previous
import functools

import jax
import jax.numpy as jnp
from jax import lax
from jax.experimental import pallas as pl
from jax.experimental.pallas import tpu as pltpu
from jax.experimental.pallas import tpu_sc as plsc

N = 10000
E = 160000
G = 20
NPG = 500
SLOPE = 0.1

NC = 2
NS = 16
NW = NC * NS
EPW = E // NW
NP = 10240
RPS = NP // NS
ZR = 128
EBW = 500
NTR = E // EBW
TPT = NTR // NW


def _lrelu(v):
    return jnp.where(v >= 0, v, SLOPE * v)


def _zero_acc(zbuf, acc, s, d):
    def zrow(i, carry):
        for j in range(d // 16):
            zbuf[i, pl.ds(j * 16, 16)] = jnp.zeros((16,), jnp.float32)
        return carry

    lax.fori_loop(0, ZR, zrow, 0)
    for j in range(RPS // ZR):
        pltpu.sync_copy(zbuf, acc.at[pl.ds(s * RPS + j * ZR, ZR)])


def _flush_acc(acc, out_hbm, c, s):
    pltpu.sync_copy(acc.at[pl.ds(s * RPS, RPS)],
                    out_hbm.at[pl.ds(c * NP + s * RPS, RPS)])


def _make_scatter(d):
    mesh = plsc.VectorSubcoreMesh(core_axis_name="c", subcore_axis_name="s")

    @functools.partial(
        pl.kernel,
        out_type=jax.ShapeDtypeStruct((2 * NP, d), jnp.float32),
        mesh=mesh,
        compiler_params=pltpu.CompilerParams(use_tc_tiling_on_sc=False),
        scratch_types=[
            pltpu.VMEM((TPT, EBW), jnp.int32),
            pltpu.VMEM((TPT, EBW), jnp.int32),
            pltpu.VMEM((EBW, d), jnp.float32),
            pltpu.VMEM((EBW, d), jnp.float32),
            pltpu.VMEM((ZR, d), jnp.float32),
            pltpu.VMEM_SHARED((NP, d), jnp.float32),
            pltpu.SemaphoreType.DMA,
            pltpu.SemaphoreType.DMA,
            pltpu.SemaphoreType.DMA,
            pltpu.SemaphoreType.DMA,
        ],
    )
    def k(g_hbm, rowm_hbm, colm_hbm, out_hbm,
          rowb, colb, msg0, msg1, zbuf, acc, gsem0, gsem1, ssem0, ssem1):
        c = lax.axis_index("c")
        s = lax.axis_index("s")
        wid = s * NC + c
        msg = (msg0, msg1)
        gsem = (gsem0, gsem1)
        ssem = (ssem0, ssem1)

        _zero_acc(zbuf, acc, s, d)

        br = wid * TPT
        pltpu.sync_copy(rowm_hbm.at[pl.ds(br, TPT)], rowb)
        pltpu.sync_copy(colm_hbm.at[pl.ds(br, TPT)], colb)
        plsc.subcore_barrier()

        def start_g(st, b):
            pltpu.async_copy(g_hbm.at[rowb.at[st]], msg[b], gsem[b])

        def start_s(st, b):
            pltpu.async_copy(msg[b], acc.at[colb.at[st]], ssem[b], add=True)

        def wait(b, sems):
            pltpu.make_async_copy(g_hbm.at[pl.ds(0, EBW)], msg[b],
                                  sems[b]).wait()

        start_g(0, 0)

        def pair(t, carry):
            for b in range(2):
                st = 2 * t + b
                wait(b, gsem)

                @pl.when(st + 1 < TPT)
                def _():
                    @pl.when(st >= 1)
                    def _():
                        wait(1 - b, ssem)

                    start_g(st + 1, 1 - b)

                start_s(st, b)
            return carry

        lax.fori_loop(0, TPT // 2, pair, 0)
        wait(0, ssem)
        wait(1, ssem)

        plsc.subcore_barrier()
        _flush_acc(acc, out_hbm, c, s)

    return k


def _make_deg():
    d = 16
    mesh = plsc.VectorSubcoreMesh(core_axis_name="c", subcore_axis_name="s")

    @functools.partial(
        pl.kernel,
        out_type=jax.ShapeDtypeStruct((2 * NP, d), jnp.float32),
        mesh=mesh,
        compiler_params=pltpu.CompilerParams(use_tc_tiling_on_sc=False),
        scratch_types=[
            pltpu.VMEM((TPT, EBW), jnp.int32),
            pltpu.VMEM((EBW, d), jnp.float32),
            pltpu.VMEM((ZR, d), jnp.float32),
            pltpu.VMEM_SHARED((NP, d), jnp.float32),
        ],
    )
    def k(colm_hbm, out_hbm, colb, ones, zbuf, acc):
        c = lax.axis_index("c")
        s = lax.axis_index("s")
        wid = s * NC + c

        def orow(i, carry):
            ones[i, pl.ds(0, 16)] = jnp.ones((16,), jnp.float32)
            return carry

        lax.fori_loop(0, EBW, orow, 0)
        _zero_acc(zbuf, acc, s, d)

        br = wid * TPT
        pltpu.sync_copy(colm_hbm.at[pl.ds(br, TPT)], colb)
        plsc.subcore_barrier()

        def step(st, carry):
            pltpu.sync_copy(ones, acc.at[colb.at[st]], add=True)
            return carry

        lax.fori_loop(0, TPT, step, 0)

        plsc.subcore_barrier()
        _flush_acc(acc, out_hbm, c, s)

    return k


_deg = _make_deg()
_scatter32 = _make_scatter(32)
_scatter64 = _make_scatter(64)


_R = 2000


def _prep_body(p0, p1, x, w, dinv_ref, g_ref):
    deg = p0[:, 0:1] + p1[:, 0:1]
    dinv = jnp.where(deg > 0, lax.rsqrt(deg), 0.0)
    dinv_ref[...] = dinv
    g_ref[...] = dinv * jnp.dot(x[...], w[...],
                                preferred_element_type=jnp.float32)


def _prep(p0, p1, x, w):
    din, dw = x.shape[1], w.shape[1]
    return pl.pallas_call(
        _prep_body,
        grid=(N // _R,),
        in_specs=[
            pl.BlockSpec((_R, 16), lambda i: (i, 0)),
            pl.BlockSpec((_R, 16), lambda i: (i, 0)),
            pl.BlockSpec((_R, din), lambda i: (i, 0)),
            pl.BlockSpec((din, dw), lambda i: (0, 0)),
        ],
        out_specs=[
            pl.BlockSpec((_R, 1), lambda i: (i, 0)),
            pl.BlockSpec((_R, dw), lambda i: (i, 0)),
        ],
        out_shape=[
            jax.ShapeDtypeStruct((N, 1), jnp.float32),
            jax.ShapeDtypeStruct((N, dw), jnp.float32),
        ],
    )(p0, p1, x, w)


def _mid_body(q0, q1, dinv, b, w, g_ref):
    a = dinv[...] * (q0[...] + q1[...]) + b[...]
    a = _lrelu(a)
    g_ref[...] = dinv[...] * jnp.dot(a, w[...],
                                     preferred_element_type=jnp.float32)


def _mid(q0, q1, dinv, b, w):
    d, dw = w.shape
    return pl.pallas_call(
        _mid_body,
        grid=(N // _R,),
        in_specs=[
            pl.BlockSpec((_R, d), lambda i: (i, 0)),
            pl.BlockSpec((_R, d), lambda i: (i, 0)),
            pl.BlockSpec((_R, 1), lambda i: (i, 0)),
            pl.BlockSpec((1, d), lambda i: (0, 0)),
            pl.BlockSpec((d, dw), lambda i: (0, 0)),
        ],
        out_specs=pl.BlockSpec((_R, dw), lambda i: (i, 0)),
        out_shape=jax.ShapeDtypeStruct((N, dw), jnp.float32),
    )(q0, q1, dinv, b, w)


def _final_body(q0, q1, dinv, b2, m0, mb0, m1, mb1, m2, mb2, h_ref, vp_ref):
    h = dinv[...] * (q0[...] + q1[...]) + b2[...]
    h_ref[...] = h
    t = _lrelu(jnp.dot(h, m0[...], preferred_element_type=jnp.float32) + mb0[...])
    t = _lrelu(jnp.dot(t, m1[...], preferred_element_type=jnp.float32) + mb1[...])
    vp_ref[...] = jnp.dot(t, m2[...], preferred_element_type=jnp.float32) + mb2[...]


def _final(q0, q1, dinv, b2, m0, mb0, m1, mb1, m2, mb2):
    return pl.pallas_call(
        _final_body,
        grid=(N // _R,),
        in_specs=[
            pl.BlockSpec((_R, 64), lambda i: (i, 0)),
            pl.BlockSpec((_R, 64), lambda i: (i, 0)),
            pl.BlockSpec((_R, 1), lambda i: (i, 0)),
            pl.BlockSpec((1, 64), lambda i: (0, 0)),
            pl.BlockSpec((64, 64), lambda i: (0, 0)),
            pl.BlockSpec((1, 64), lambda i: (0, 0)),
            pl.BlockSpec((64, 64), lambda i: (0, 0)),
            pl.BlockSpec((1, 64), lambda i: (0, 0)),
            pl.BlockSpec((64, 4), lambda i: (0, 0)),
            pl.BlockSpec((1, 4), lambda i: (0, 0)),
        ],
        out_specs=[
            pl.BlockSpec((_R, 64), lambda i: (i, 0)),
            pl.BlockSpec((_R, 4), lambda i: (i, 0)),
        ],
        out_shape=[
            jax.ShapeDtypeStruct((N, 64), jnp.float32),
            jax.ShapeDtypeStruct((N, 4), jnp.float32),
        ],
    )(q0, q1, dinv, b2, m0, mb0, m1, mb1, m2, mb2)


_OB = 400
_VD = NPG * 4


def _outer_body(vr, vc, o_ref):
    o_ref[0] = vr[0] * vc[0]


def _outer(vcol, vrow):
    return pl.pallas_call(
        _outer_body,
        grid=(G,),
        in_specs=[
            pl.BlockSpec((1, _VD, 1), lambda k: (k, 0, 0)),
            pl.BlockSpec((1, 1, _VD), lambda k: (k, 0, 0)),
        ],
        out_specs=pl.BlockSpec((1, _VD, _VD), lambda k: (k, 0, 0)),
        out_shape=jax.ShapeDtypeStruct((G, _VD, _VD), jnp.float32),
        compiler_params=pltpu.CompilerParams(
            vmem_limit_bytes=100 * 1024 * 1024),
    )(vcol, vrow)


@jax.jit
def kernel(x, edge_index, ptr, W0, b0, W1, b1, W2, b2, M0, mb0, M1, mb1, M2, mb2):
    rowm = edge_index[0].reshape(NTR, EBW)
    colm = edge_index[1].reshape(NTR, EBW)

    degp = _deg(colm)
    dinv, g0 = _prep(degp[:N], degp[NP:NP + N], x, W0)

    p = _scatter32(g0, rowm, colm)
    g1 = _mid(p[:N], p[NP:NP + N], dinv, b0.reshape(1, -1), W1)

    p = _scatter32(g1, rowm, colm)
    g2 = _mid(p[:N], p[NP:NP + N], dinv, b1.reshape(1, -1), W2)

    p = _scatter64(g2, rowm, colm)
    h, vp = _final(p[:N], p[NP:NP + N], dinv, b2.reshape(1, -1),
                   M0, mb0.reshape(1, -1), M1, mb1.reshape(1, -1),
                   M2, mb2.reshape(1, -1))

    Xs = _outer(vp.reshape(G, _VD, 1), vp.reshape(G, 1, _VD))
    return h, Xs
